# baseline (speedup 1.0000x reference)
import functools

import jax
import jax.numpy as jnp
from jax import lax
from jax.experimental import pallas as pl
from jax.experimental.pallas import tpu as pltpu
from jax.experimental.pallas import tpu_sc as plsc

N = 10000
E = 320000
D = 128
G = 64
EPS = 1e-5

NC = 2
NS = 16
DH = D // NC
E_PER_TILE = E // NS
C = 80
NCHUNK = E_PER_TILE // C
NP = 10000
ROWS_PER_TILE = NP // NS
CW = 16


@functools.lru_cache(maxsize=None)
def _get_mesh():
    return plsc.VectorSubcoreMesh(
        core_axis_name="c", subcore_axis_name="s",
        num_cores=NC, num_subcores=NS)


def _sc_agg_body(with_count, x2_hbm, src2_hbm, dst2_hbm, out_agg, out_cnt,
                 src_all, dst_all, rows, ones_v, zrows, zc, agg_sh, cnt_sh,
                 gsem, ssem, csem):
    c = lax.axis_index("c")
    s = lax.axis_index("s")

    zvec = jnp.zeros((16,), jnp.float32)

    def zrow_body(i, _):
        for j in range(DH // 16):
            zrows[i, pl.ds(j * 16, 16)] = zvec
        return 0
    lax.fori_loop(0, zrows.shape[0], zrow_body, 0)
    zchunk = zrows.shape[0]
    for k in range(ROWS_PER_TILE // zchunk):
        pltpu.sync_copy(zrows,
                        agg_sh.at[pl.ds(s * ROWS_PER_TILE + k * zchunk, zchunk)])
    if with_count:
        def zc_body(i, _):
            zc[i, :] = zvec
            return 0
        lax.fori_loop(0, ROWS_PER_TILE, zc_body, 0)
        pltpu.sync_copy(zc, cnt_sh.at[pl.ds(s * ROWS_PER_TILE, ROWS_PER_TILE)])

        ovec = jnp.ones((16,), jnp.float32)
        def ones_body(i, _):
            ones_v[i, :] = ovec
            return 0
        lax.fori_loop(0, C, ones_body, 0)

    plsc.subcore_barrier()

    pltpu.sync_copy(src2_hbm.at[pl.ds(s * NCHUNK, NCHUNK)], src_all)
    pltpu.sync_copy(dst2_hbm.at[pl.ds(s * NCHUNK, NCHUNK)], dst_all)

    def gather_start(j):
        pltpu.async_copy(x2_hbm.at[c].at[src_all.at[j]],
                         rows.at[j % 4], gsem)

    def gather_wait(j):
        pltpu.make_async_copy(x2_hbm.at[c].at[src_all.at[0]],
                              rows.at[j % 4], gsem).wait()

    def scatter_wait(j):
        pltpu.make_async_copy(rows.at[j % 4], agg_sh.at[pl.ds(0, C)],
                              ssem.at[j % 4]).wait()

    gather_start(0)

    def body(j, _):
        @pl.when(j >= 3)
        def _():
            scatter_wait(j - 3)
        if with_count:
            @pl.when(jnp.logical_and(j % 2 == c, j >= 2))
            def _():
                pltpu.make_async_copy(ones_v, cnt_sh.at[pl.ds(0, C)],
                                      csem).wait()

        gather_wait(j)

        @pl.when(j + 1 < NCHUNK)
        def _():
            gather_start(j + 1)

        pltpu.async_copy(rows.at[j % 4], agg_sh.at[dst_all.at[j]],
                         ssem.at[j % 4], add=True)
        if with_count:
            @pl.when(j % 2 == c)
            def _():
                pltpu.async_copy(ones_v, cnt_sh.at[dst_all.at[j]],
                                 csem, add=True)
        return 0
    lax.fori_loop(0, NCHUNK, body, 0)

    scatter_wait(NCHUNK - 3)
    scatter_wait(NCHUNK - 2)
    scatter_wait(NCHUNK - 1)
    if with_count:
        pltpu.make_async_copy(ones_v, cnt_sh.at[pl.ds(0, C)], csem).wait()

    plsc.subcore_barrier()

    r0 = s * ROWS_PER_TILE
    pltpu.sync_copy(agg_sh.at[pl.ds(r0, ROWS_PER_TILE)],
                    out_agg.at[c, pl.ds(r0, ROWS_PER_TILE)])
    if with_count:
        pltpu.sync_copy(cnt_sh.at[pl.ds(r0, ROWS_PER_TILE)],
                        out_cnt.at[c, pl.ds(r0, ROWS_PER_TILE)])


@functools.lru_cache(maxsize=None)
def _make_sc_agg(with_count):
    out_type = [jax.ShapeDtypeStruct((NC, NP, DH), jnp.float32)]
    scratch = [
        pltpu.VMEM((NCHUNK, C), jnp.int32),
        pltpu.VMEM((NCHUNK, C), jnp.int32),
        pltpu.VMEM((4, C, DH), jnp.float32),
        pltpu.VMEM((C, CW), jnp.float32),
        pltpu.VMEM((125, DH), jnp.float32),
        pltpu.VMEM((ROWS_PER_TILE, CW), jnp.float32),
        pltpu.VMEM_SHARED((NP, DH), jnp.float32),
        pltpu.VMEM_SHARED((NP, CW), jnp.float32),
        pltpu.SemaphoreType.DMA,
        pltpu.SemaphoreType.DMA((4,)),
        pltpu.SemaphoreType.DMA,
    ]
    if with_count:
        out_type.append(jax.ShapeDtypeStruct((NC, NP, CW), jnp.float32))

        def body(x2_hbm, src2_hbm, dst2_hbm, out_agg, out_cnt, *scr):
            _sc_agg_body(True, x2_hbm, src2_hbm, dst2_hbm, out_agg, out_cnt,
                         *scr)
    else:
        def body(x2_hbm, src2_hbm, dst2_hbm, out_agg, *scr):
            _sc_agg_body(False, x2_hbm, src2_hbm, dst2_hbm, out_agg, None,
                         *scr)

    return pl.kernel(body, out_type=out_type, mesh=_get_mesh(),
                     scratch_types=scratch,
                     compiler_params=pltpu.CompilerParams(
                         use_tc_tiling_on_sc=False),
                     name="sc_agg_cnt" if with_count else "sc_agg")


def _tc_layer0_body(parts, cnt2, x, Wlt, bl, Wrt, g, be, outs):
    agg = jnp.concatenate([parts[0][:N], parts[1][:N]], axis=1)
    cnt = cnt2[0][:N, 0:1] + cnt2[1][:N, 0:1]
    mean = agg / jnp.maximum(cnt, 1.0)
    h = (jnp.dot(mean, Wlt[...], preferred_element_type=jnp.float32) + bl[...]
         + jnp.dot(x[...], Wrt[...], preferred_element_type=jnp.float32))
    mu = jnp.mean(h, axis=0, keepdims=True)
    var = jnp.mean((h - mu) ** 2, axis=0, keepdims=True)
    hn = (h - mu) / jnp.sqrt(var + EPS) * g[...] + be[...]
    hr = jnp.maximum(hn, 0.0)
    outs[0] = hr[:, :DH]
    outs[1] = hr[:, DH:]


def _tc_layer1_body(parts, cnt2, xs, Wlt, bl, Wrt, g, be, batch, out):
    agg = jnp.concatenate([parts[0][:N], parts[1][:N]], axis=1)
    cnt = cnt2[0][:N, 0:1] + cnt2[1][:N, 0:1]
    mean = agg / jnp.maximum(cnt, 1.0)
    x = jnp.concatenate([xs[0], xs[1]], axis=1)
    h = (jnp.dot(mean, Wlt[...], preferred_element_type=jnp.float32) + bl[...]
         + jnp.dot(x, Wrt[...], preferred_element_type=jnp.float32))
    mu = jnp.mean(h, axis=0, keepdims=True)
    var = jnp.mean((h - mu) ** 2, axis=0, keepdims=True)
    hn = (h - mu) / jnp.sqrt(var + EPS) * g[...] + be[...]
    hr = jnp.maximum(hn, 0.0)
    ids = lax.broadcasted_iota(jnp.int32, (G, N), 0)
    onehot = (batch[...] == ids).astype(jnp.float32)
    sums = jnp.dot(onehot, hr, preferred_element_type=jnp.float32)
    cg = jnp.sum(onehot, axis=1, keepdims=True)
    out[...] = sums / jnp.maximum(cg, 1.0)


_tc_layer0 = pl.pallas_call(
    _tc_layer0_body,
    out_shape=jax.ShapeDtypeStruct((NC, N, DH), jnp.float32),
    name="tc_layer0",
)

_tc_layer1 = pl.pallas_call(
    _tc_layer1_body,
    out_shape=jax.ShapeDtypeStruct((G, D), jnp.float32),
    name="tc_layer1",
)


def kernel(x, edge_index, batch, W_l0, b_l0, W_r0, g0, be0,
           W_l1, b_l1, W_r1, g1, be1):
    src2 = edge_index[0].reshape(E // C, C)
    dst2 = edge_index[1].reshape(E // C, C)
    x2 = jnp.stack([x[:, :DH], x[:, DH:]], axis=0)
    agg0, cnt = _make_sc_agg(True)(x2, src2, dst2)
    h0s = _tc_layer0(agg0, cnt, x, W_l0.T, b_l0.reshape(1, D), W_r0.T,
                     g0.reshape(1, D), be0.reshape(1, D))
    agg1, = _make_sc_agg(False)(h0s, src2, dst2)
    out = _tc_layer1(agg1, cnt, h0s, W_l1.T, b_l1.reshape(1, D), W_r1.T,
                     g1.reshape(1, D), be1.reshape(1, D),
                     batch.reshape(1, N))
    return out


# speedup vs baseline: 1.4552x; 1.4552x over previous
import functools

import jax
import jax.numpy as jnp
from jax import lax
from jax.experimental import pallas as pl
from jax.experimental.pallas import tpu as pltpu
from jax.experimental.pallas import tpu_sc as plsc

N = 10000
E = 320000
D = 128
G = 64
EPS = 1e-5

NC = 2
NS = 16
DH = D // NC
E_PER_TILE = E // NS
C = 80
NCHUNK = E_PER_TILE // C
NP = 10000
ROWS_PER_TILE = NP // NS
CW = 16
DEPTH = 2
RING = 4


@functools.lru_cache(maxsize=None)
def _get_mesh():
    return plsc.VectorSubcoreMesh(
        core_axis_name="c", subcore_axis_name="s",
        num_cores=NC, num_subcores=NS)


def _sc_agg_body(with_count, x2_hbm, src2_hbm, dst2_hbm, out_agg, out_cnt,
                 src_all, dst_all, rows, ones_v, zrows, zc, agg_sh, cnt_sh,
                 gsem, ssem, csem):
    c = lax.axis_index("c")
    s = lax.axis_index("s")

    zvec = jnp.zeros((16,), jnp.float32)

    def zrow_body(i, _):
        for j in range(DH // 16):
            zrows[i, pl.ds(j * 16, 16)] = zvec
        return 0
    lax.fori_loop(0, zrows.shape[0], zrow_body, 0)
    zchunk = zrows.shape[0]
    for k in range(ROWS_PER_TILE // zchunk):
        pltpu.sync_copy(zrows,
                        agg_sh.at[pl.ds(s * ROWS_PER_TILE + k * zchunk, zchunk)])
    if with_count:
        def zc_body(i, _):
            zc[i, :] = zvec
            return 0
        lax.fori_loop(0, ROWS_PER_TILE, zc_body, 0)
        pltpu.sync_copy(zc, cnt_sh.at[pl.ds(s * ROWS_PER_TILE, ROWS_PER_TILE)])

        ovec = jnp.ones((16,), jnp.float32)
        def ones_body(i, _):
            ones_v[i, :] = ovec
            return 0
        lax.fori_loop(0, C, ones_body, 0)

    plsc.subcore_barrier()

    pltpu.sync_copy(src2_hbm.at[pl.ds(s * NCHUNK, NCHUNK)], src_all)
    pltpu.sync_copy(dst2_hbm.at[pl.ds(s * NCHUNK, NCHUNK)], dst_all)

    depth = DEPTH
    ring = RING

    def gather_start(j):
        pltpu.async_copy(x2_hbm.at[c].at[src_all.at[j]],
                         rows.at[j % ring], gsem.at[j % ring])

    def gather_wait(j):
        pltpu.make_async_copy(x2_hbm.at[c].at[src_all.at[0]],
                              rows.at[j % ring], gsem.at[j % ring]).wait()

    def scatter_wait(j):
        pltpu.make_async_copy(rows.at[j % ring], agg_sh.at[pl.ds(0, C)],
                              ssem.at[j % ring]).wait()

    for jj in range(depth):
        gather_start(jj)

    def body(j, _):
        @pl.when(j >= ring - depth)
        def _():
            scatter_wait(j - (ring - depth))
        if with_count:
            @pl.when(jnp.logical_and(j % 2 == c, j >= 2))
            def _():
                pltpu.make_async_copy(ones_v, cnt_sh.at[pl.ds(0, C)],
                                      csem).wait()

        gather_wait(j)

        @pl.when(j + depth < NCHUNK)
        def _():
            gather_start(j + depth)

        pltpu.async_copy(rows.at[j % ring], agg_sh.at[dst_all.at[j]],
                         ssem.at[j % ring], add=True)
        if with_count:
            @pl.when(j % 2 == c)
            def _():
                pltpu.async_copy(ones_v, cnt_sh.at[dst_all.at[j]],
                                 csem, add=True)
        return 0
    lax.fori_loop(0, NCHUNK, body, 0)

    for jj in range(ring - depth):
        scatter_wait(NCHUNK - (ring - depth) + jj)
    if with_count:
        pltpu.make_async_copy(ones_v, cnt_sh.at[pl.ds(0, C)], csem).wait()

    plsc.subcore_barrier()

    r0 = s * ROWS_PER_TILE
    pltpu.sync_copy(agg_sh.at[pl.ds(r0, ROWS_PER_TILE)],
                    out_agg.at[c, pl.ds(r0, ROWS_PER_TILE)])
    if with_count:
        pltpu.sync_copy(cnt_sh.at[pl.ds(r0, ROWS_PER_TILE)],
                        out_cnt.at[c, pl.ds(r0, ROWS_PER_TILE)])


@functools.lru_cache(maxsize=None)
def _make_sc_agg(with_count):
    out_type = [jax.ShapeDtypeStruct((NC, NP, DH), jnp.float32)]
    scratch = [
        pltpu.VMEM((NCHUNK, C), jnp.int32),
        pltpu.VMEM((NCHUNK, C), jnp.int32),
        pltpu.VMEM((RING, C, DH), jnp.float32),
        pltpu.VMEM((C, CW), jnp.float32),
        pltpu.VMEM((125, DH), jnp.float32),
        pltpu.VMEM((ROWS_PER_TILE, CW), jnp.float32),
        pltpu.VMEM_SHARED((NP, DH), jnp.float32),
        pltpu.VMEM_SHARED((NP, CW), jnp.float32),
        pltpu.SemaphoreType.DMA((RING,)),
        pltpu.SemaphoreType.DMA((RING,)),
        pltpu.SemaphoreType.DMA,
    ]
    if with_count:
        out_type.append(jax.ShapeDtypeStruct((NC, NP, CW), jnp.float32))

        def body(x2_hbm, src2_hbm, dst2_hbm, out_agg, out_cnt, *scr):
            _sc_agg_body(True, x2_hbm, src2_hbm, dst2_hbm, out_agg, out_cnt,
                         *scr)
    else:
        def body(x2_hbm, src2_hbm, dst2_hbm, out_agg, *scr):
            _sc_agg_body(False, x2_hbm, src2_hbm, dst2_hbm, out_agg, None,
                         *scr)

    return pl.kernel(body, out_type=out_type, mesh=_get_mesh(),
                     scratch_types=scratch,
                     compiler_params=pltpu.CompilerParams(
                         use_tc_tiling_on_sc=False),
                     name="sc_agg_cnt" if with_count else "sc_agg")


def _tc_layer0_body(parts, cnt2, x, Wlt, bl, Wrt, g, be, outs):
    agg = jnp.concatenate([parts[0][:N], parts[1][:N]], axis=1)
    cnt = cnt2[0][:N, 0:1] + cnt2[1][:N, 0:1]
    mean = agg / jnp.maximum(cnt, 1.0)
    h = (jnp.dot(mean, Wlt[...], preferred_element_type=jnp.float32) + bl[...]
         + jnp.dot(x[...], Wrt[...], preferred_element_type=jnp.float32))
    mu = jnp.mean(h, axis=0, keepdims=True)
    var = jnp.mean((h - mu) ** 2, axis=0, keepdims=True)
    hn = (h - mu) / jnp.sqrt(var + EPS) * g[...] + be[...]
    hr = jnp.maximum(hn, 0.0)
    outs[0] = hr[:, :DH]
    outs[1] = hr[:, DH:]


def _tc_layer1_body(parts, cnt2, xs, Wlt, bl, Wrt, g, be, batch, out):
    agg = jnp.concatenate([parts[0][:N], parts[1][:N]], axis=1)
    cnt = cnt2[0][:N, 0:1] + cnt2[1][:N, 0:1]
    mean = agg / jnp.maximum(cnt, 1.0)
    x = jnp.concatenate([xs[0], xs[1]], axis=1)
    h = (jnp.dot(mean, Wlt[...], preferred_element_type=jnp.float32) + bl[...]
         + jnp.dot(x, Wrt[...], preferred_element_type=jnp.float32))
    mu = jnp.mean(h, axis=0, keepdims=True)
    var = jnp.mean((h - mu) ** 2, axis=0, keepdims=True)
    hn = (h - mu) / jnp.sqrt(var + EPS) * g[...] + be[...]
    hr = jnp.maximum(hn, 0.0)
    ids = lax.broadcasted_iota(jnp.int32, (G, N), 0)
    onehot = (batch[...] == ids).astype(jnp.float32)
    sums = jnp.dot(onehot, hr, preferred_element_type=jnp.float32)
    cg = jnp.sum(onehot, axis=1, keepdims=True)
    out[...] = sums / jnp.maximum(cg, 1.0)


_tc_layer0 = pl.pallas_call(
    _tc_layer0_body,
    out_shape=jax.ShapeDtypeStruct((NC, N, DH), jnp.float32),
    name="tc_layer0",
)

_tc_layer1 = pl.pallas_call(
    _tc_layer1_body,
    out_shape=jax.ShapeDtypeStruct((G, D), jnp.float32),
    name="tc_layer1",
)


def kernel(x, edge_index, batch, W_l0, b_l0, W_r0, g0, be0,
           W_l1, b_l1, W_r1, g1, be1):
    src2 = edge_index[0].reshape(E // C, C)
    dst2 = edge_index[1].reshape(E // C, C)
    x2 = jnp.stack([x[:, :DH], x[:, DH:]], axis=0)
    agg0, cnt = _make_sc_agg(True)(x2, src2, dst2)
    h0s = _tc_layer0(agg0, cnt, x, W_l0.T, b_l0.reshape(1, D), W_r0.T,
                     g0.reshape(1, D), be0.reshape(1, D))
    agg1, = _make_sc_agg(False)(h0s, src2, dst2)
    out = _tc_layer1(agg1, cnt, h0s, W_l1.T, b_l1.reshape(1, D), W_r1.T,
                     g1.reshape(1, D), be1.reshape(1, D),
                     batch.reshape(1, N))
    return out


# speedup vs baseline: 1.5385x; 1.0572x over previous
import functools

import jax
import jax.numpy as jnp
from jax import lax
from jax.experimental import pallas as pl
from jax.experimental.pallas import tpu as pltpu
from jax.experimental.pallas import tpu_sc as plsc

N = 10000
E = 320000
D = 128
G = 64
EPS = 1e-5

NC = 2
NS = 16
DH = D // NC
E_PER_TILE = E // NS
C = 80
NCHUNK = E_PER_TILE // C
NP = 10000
ROWS_PER_TILE = NP // NS
CW = 16
DEPTH = 3
RING = 6


@functools.lru_cache(maxsize=None)
def _get_mesh():
    return plsc.VectorSubcoreMesh(
        core_axis_name="c", subcore_axis_name="s",
        num_cores=NC, num_subcores=NS)


def _sc_agg_body(with_count, x2_hbm, src2_hbm, dst2_hbm, out_agg, out_cnt,
                 src_all, dst_all, rows, ones_v, zrows, zc, agg_sh, cnt_sh,
                 gsem, ssem, csem):
    c = lax.axis_index("c")
    s = lax.axis_index("s")

    zvec = jnp.zeros((16,), jnp.float32)

    def zrow_body(i, _):
        for j in range(DH // 16):
            zrows[i, pl.ds(j * 16, 16)] = zvec
        return 0
    lax.fori_loop(0, zrows.shape[0], zrow_body, 0)
    zchunk = zrows.shape[0]
    for k in range(ROWS_PER_TILE // zchunk):
        pltpu.sync_copy(zrows,
                        agg_sh.at[pl.ds(s * ROWS_PER_TILE + k * zchunk, zchunk)])
    if with_count:
        def zc_body(i, _):
            zc[i, :] = zvec
            return 0
        lax.fori_loop(0, ROWS_PER_TILE, zc_body, 0)
        pltpu.sync_copy(zc, cnt_sh.at[pl.ds(s * ROWS_PER_TILE, ROWS_PER_TILE)])

        ovec = jnp.ones((16,), jnp.float32)
        def ones_body(i, _):
            ones_v[i, :] = ovec
            return 0
        lax.fori_loop(0, C, ones_body, 0)

    plsc.subcore_barrier()

    pltpu.sync_copy(src2_hbm.at[pl.ds(s * NCHUNK, NCHUNK)], src_all)
    pltpu.sync_copy(dst2_hbm.at[pl.ds(s * NCHUNK, NCHUNK)], dst_all)

    depth = 2 if with_count else DEPTH
    ring = 4 if with_count else RING

    def gather_start(j):
        pltpu.async_copy(x2_hbm.at[c].at[src_all.at[j]],
                         rows.at[j % ring], gsem.at[j % ring])

    def gather_wait(j):
        pltpu.make_async_copy(x2_hbm.at[c].at[src_all.at[0]],
                              rows.at[j % ring], gsem.at[j % ring]).wait()

    def scatter_wait(j):
        pltpu.make_async_copy(rows.at[j % ring], agg_sh.at[pl.ds(0, C)],
                              ssem.at[j % ring]).wait()

    for jj in range(depth):
        gather_start(jj)

    def body(j, _):
        @pl.when(j >= ring - depth)
        def _():
            scatter_wait(j - (ring - depth))
        if with_count:
            @pl.when(jnp.logical_and(j % 2 == c, j >= 2))
            def _():
                pltpu.make_async_copy(ones_v, cnt_sh.at[pl.ds(0, C)],
                                      csem).wait()

        gather_wait(j)

        @pl.when(j + depth < NCHUNK)
        def _():
            gather_start(j + depth)

        pltpu.async_copy(rows.at[j % ring], agg_sh.at[dst_all.at[j]],
                         ssem.at[j % ring], add=True)
        if with_count:
            @pl.when(j % 2 == c)
            def _():
                pltpu.async_copy(ones_v, cnt_sh.at[dst_all.at[j]],
                                 csem, add=True)
        return 0
    lax.fori_loop(0, NCHUNK, body, 0)

    for jj in range(ring - depth):
        scatter_wait(NCHUNK - (ring - depth) + jj)
    if with_count:
        pltpu.make_async_copy(ones_v, cnt_sh.at[pl.ds(0, C)], csem).wait()

    plsc.subcore_barrier()

    r0 = s * ROWS_PER_TILE
    pltpu.sync_copy(agg_sh.at[pl.ds(r0, ROWS_PER_TILE)],
                    out_agg.at[c, pl.ds(r0, ROWS_PER_TILE)])
    if with_count:
        pltpu.sync_copy(cnt_sh.at[pl.ds(r0, ROWS_PER_TILE)],
                        out_cnt.at[c, pl.ds(r0, ROWS_PER_TILE)])


@functools.lru_cache(maxsize=None)
def _make_sc_agg(with_count):
    out_type = [jax.ShapeDtypeStruct((NC, NP, DH), jnp.float32)]
    scratch = [
        pltpu.VMEM((NCHUNK, C), jnp.int32),
        pltpu.VMEM((NCHUNK, C), jnp.int32),
        pltpu.VMEM(((4 if with_count else RING), C, DH), jnp.float32),
        pltpu.VMEM((C, CW), jnp.float32),
        pltpu.VMEM((125, DH), jnp.float32),
        pltpu.VMEM((ROWS_PER_TILE, CW), jnp.float32),
        pltpu.VMEM_SHARED((NP, DH), jnp.float32),
        pltpu.VMEM_SHARED((NP, CW), jnp.float32),
        pltpu.SemaphoreType.DMA(((4 if with_count else RING),)),
        pltpu.SemaphoreType.DMA(((4 if with_count else RING),)),
        pltpu.SemaphoreType.DMA,
    ]
    if with_count:
        out_type.append(jax.ShapeDtypeStruct((NC, NP, CW), jnp.float32))

        def body(x2_hbm, src2_hbm, dst2_hbm, out_agg, out_cnt, *scr):
            _sc_agg_body(True, x2_hbm, src2_hbm, dst2_hbm, out_agg, out_cnt,
                         *scr)
    else:
        def body(x2_hbm, src2_hbm, dst2_hbm, out_agg, *scr):
            _sc_agg_body(False, x2_hbm, src2_hbm, dst2_hbm, out_agg, None,
                         *scr)

    return pl.kernel(body, out_type=out_type, mesh=_get_mesh(),
                     scratch_types=scratch,
                     compiler_params=pltpu.CompilerParams(
                         use_tc_tiling_on_sc=False),
                     name="sc_agg_cnt" if with_count else "sc_agg")


def _tc_layer0_body(parts, cnt2, x, Wlt, bl, Wrt, g, be, outs):
    agg = jnp.concatenate([parts[0][:N], parts[1][:N]], axis=1)
    cnt = cnt2[0][:N, 0:1] + cnt2[1][:N, 0:1]
    mean = agg / jnp.maximum(cnt, 1.0)
    h = (jnp.dot(mean, Wlt[...], preferred_element_type=jnp.float32) + bl[...]
         + jnp.dot(x[...], Wrt[...], preferred_element_type=jnp.float32))
    mu = jnp.mean(h, axis=0, keepdims=True)
    var = jnp.mean((h - mu) ** 2, axis=0, keepdims=True)
    hn = (h - mu) / jnp.sqrt(var + EPS) * g[...] + be[...]
    hr = jnp.maximum(hn, 0.0)
    outs[0] = hr[:, :DH]
    outs[1] = hr[:, DH:]


def _tc_layer1_body(parts, cnt2, xs, Wlt, bl, Wrt, g, be, batch, out):
    agg = jnp.concatenate([parts[0][:N], parts[1][:N]], axis=1)
    cnt = cnt2[0][:N, 0:1] + cnt2[1][:N, 0:1]
    mean = agg / jnp.maximum(cnt, 1.0)
    x = jnp.concatenate([xs[0], xs[1]], axis=1)
    h = (jnp.dot(mean, Wlt[...], preferred_element_type=jnp.float32) + bl[...]
         + jnp.dot(x, Wrt[...], preferred_element_type=jnp.float32))
    mu = jnp.mean(h, axis=0, keepdims=True)
    var = jnp.mean((h - mu) ** 2, axis=0, keepdims=True)
    hn = (h - mu) / jnp.sqrt(var + EPS) * g[...] + be[...]
    hr = jnp.maximum(hn, 0.0)
    ids = lax.broadcasted_iota(jnp.int32, (G, N), 0)
    onehot = (batch[...] == ids).astype(jnp.float32)
    sums = jnp.dot(onehot, hr, preferred_element_type=jnp.float32)
    cg = jnp.sum(onehot, axis=1, keepdims=True)
    out[...] = sums / jnp.maximum(cg, 1.0)


_tc_layer0 = pl.pallas_call(
    _tc_layer0_body,
    out_shape=jax.ShapeDtypeStruct((NC, N, DH), jnp.float32),
    name="tc_layer0",
)

_tc_layer1 = pl.pallas_call(
    _tc_layer1_body,
    out_shape=jax.ShapeDtypeStruct((G, D), jnp.float32),
    name="tc_layer1",
)


def kernel(x, edge_index, batch, W_l0, b_l0, W_r0, g0, be0,
           W_l1, b_l1, W_r1, g1, be1):
    src2 = edge_index[0].reshape(E // C, C)
    dst2 = edge_index[1].reshape(E // C, C)
    x2 = jnp.stack([x[:, :DH], x[:, DH:]], axis=0)
    agg0, cnt = _make_sc_agg(True)(x2, src2, dst2)
    h0s = _tc_layer0(agg0, cnt, x, W_l0.T, b_l0.reshape(1, D), W_r0.T,
                     g0.reshape(1, D), be0.reshape(1, D))
    agg1, = _make_sc_agg(False)(h0s, src2, dst2)
    out = _tc_layer1(agg1, cnt, h0s, W_l1.T, b_l1.reshape(1, D), W_r1.T,
                     g1.reshape(1, D), be1.reshape(1, D),
                     batch.reshape(1, N))
    return out
